# baseline (speedup 1.0000x reference)
import functools

import jax
import jax.numpy as jnp
from jax import lax
from jax.experimental import pallas as pl
from jax.experimental.pallas import tpu as pltpu
from jax.experimental.pallas import tpu_sc as plsc

NC = 2
NS = 16
NW = NC * NS
CH = 96


def _ceil_to(a: int, m: int) -> int:
    return ((a + m - 1) // m) * m


def _sc_scatter_rows(src_idx, dst_idx, table, n_out: int, d: int,
                     add: bool = True, mode: str = "overlap"):
    kc = src_idx.shape[1]
    rpt = n_out // NS
    mesh = plsc.VectorSubcoreMesh(core_axis_name="c", subcore_axis_name="s")

    @functools.partial(
        pl.kernel,
        out_type=jax.ShapeDtypeStruct((NC, n_out, d), jnp.float32),
        mesh=mesh,
        scratch_types=[
            pltpu.VMEM((kc, CH), jnp.int32),
            pltpu.VMEM((kc, CH), jnp.int32),
            pltpu.VMEM((CH, d), jnp.float32),
            pltpu.VMEM((CH, d), jnp.float32),
            pltpu.VMEM_SHARED((n_out, d), jnp.float32),
            pltpu.SemaphoreType.DMA,
            pltpu.SemaphoreType.DMA,
        ],
        compiler_params=pltpu.CompilerParams(use_tc_tiling_on_sc=False),
    )
    def k(src_hbm, dst_hbm, tab_hbm, zero_hbm, out_hbm, sv, dv, ra, rb, acc,
          sa, sb):
        c = lax.axis_index("c")
        s = lax.axis_index("s")
        wid = c * NS + s
        pltpu.sync_copy(zero_hbm.at[pl.ds(s * rpt, rpt)],
                        acc.at[pl.ds(s * rpt, rpt)])
        pltpu.sync_copy(src_hbm.at[wid], sv)
        pltpu.sync_copy(dst_hbm.at[wid], dv)
        plsc.subcore_barrier()

        if mode == "overlap":
            pltpu.async_copy(tab_hbm.at[sv.at[0]], ra, sa)

            @pl.loop(0, kc, step=2)
            def _(j):
                pltpu.make_async_copy(tab_hbm.at[sv.at[j]], ra, sa).wait()
                pltpu.async_copy(tab_hbm.at[sv.at[j + 1]], rb, sb)
                pltpu.sync_copy(ra, acc.at[dv.at[j]], add=add)
                pltpu.make_async_copy(tab_hbm.at[sv.at[j + 1]], rb, sb).wait()

                @pl.when(j + 2 < kc)
                def _():
                    pltpu.async_copy(tab_hbm.at[sv.at[j + 2]], ra, sa)

                pltpu.sync_copy(rb, acc.at[dv.at[j + 1]], add=add)
        elif mode == "serial":
            @pl.loop(0, kc)
            def _(j):
                pltpu.sync_copy(tab_hbm.at[sv.at[j]], ra)
                pltpu.sync_copy(ra, acc.at[dv.at[j]], add=add)
        else:
            @pl.loop(0, kc, step=2)
            def _(j):
                pltpu.async_copy(tab_hbm.at[sv.at[j]], ra, sa)
                pltpu.async_copy(tab_hbm.at[sv.at[j + 1]], rb, sb)
                pltpu.make_async_copy(tab_hbm.at[sv.at[j]], ra, sa).wait()
                pltpu.make_async_copy(tab_hbm.at[sv.at[j + 1]], rb, sb).wait()
                pltpu.sync_copy(ra, acc.at[dv.at[j]], add=add)
                pltpu.sync_copy(rb, acc.at[dv.at[j + 1]], add=add)

        plsc.subcore_barrier()
        pltpu.sync_copy(acc.at[pl.ds(s * rpt, rpt)],
                        out_hbm.at[c, pl.ds(s * rpt, rpt)])

    zero = jnp.zeros((n_out, d), jnp.float32)
    return k(src_idx, dst_idx, table, zero)


def _tc_stage1(x, w1, deg0, deg1, bn: int):
    n, dx = x.shape
    h = w1.shape[1]

    def body(x_ref, w_ref, d0_ref, d1_ref, g_ref, di_ref):
        deg = d0_ref[:, 0:1] + d1_ref[:, 0:1] + 1.0
        dinv = lax.rsqrt(deg)
        hh = jnp.dot(x_ref[...], w_ref[...], preferred_element_type=jnp.float32)
        g_ref[...] = hh * dinv
        di_ref[...] = jnp.broadcast_to(dinv, di_ref.shape)

    return pl.pallas_call(
        body,
        grid=(n // bn,),
        in_specs=[
            pl.BlockSpec((bn, dx), lambda i: (i, 0)),
            pl.BlockSpec((dx, h), lambda i: (0, 0)),
            pl.BlockSpec((bn, 16), lambda i: (i, 0)),
            pl.BlockSpec((bn, 16), lambda i: (i, 0)),
        ],
        out_specs=[
            pl.BlockSpec((bn, h), lambda i: (i, 0)),
            pl.BlockSpec((bn, 16), lambda i: (i, 0)),
        ],
        out_shape=[
            jax.ShapeDtypeStruct((n, h), jnp.float32),
            jax.ShapeDtypeStruct((n, 16), jnp.float32),
        ],
    )(x, w1, deg0, deg1)


def _tc_stage2(p0, p1, dinv16, b1, w2p, bn: int):
    n, h = p0.shape
    cp = w2p.shape[1]

    def body(p0_ref, p1_ref, di_ref, b_ref, w_ref, g_ref):
        di = di_ref[:, 0:1]
        a = jnp.maximum((p0_ref[...] + p1_ref[...]) * di + b_ref[...], 0.0)
        hh = jnp.dot(a, w_ref[...], preferred_element_type=jnp.float32)
        g_ref[...] = hh * di

    return pl.pallas_call(
        body,
        grid=(n // bn,),
        in_specs=[
            pl.BlockSpec((bn, h), lambda i: (i, 0)),
            pl.BlockSpec((bn, h), lambda i: (i, 0)),
            pl.BlockSpec((bn, 16), lambda i: (i, 0)),
            pl.BlockSpec((1, h), lambda i: (0, 0)),
            pl.BlockSpec((h, cp), lambda i: (0, 0)),
        ],
        out_specs=pl.BlockSpec((bn, cp), lambda i: (i, 0)),
        out_shape=jax.ShapeDtypeStruct((n, cp), jnp.float32),
    )(p0, p1, dinv16, b1, w2p)


def _tc_stage3(q0, q1, dinv16, b2p, bn: int):
    n, cp = q0.shape

    def body(q0_ref, q1_ref, di_ref, b_ref, o_ref):
        di = di_ref[:, 0:1]
        o_ref[...] = (q0_ref[...] + q1_ref[...]) * di + b_ref[...]

    return pl.pallas_call(
        body,
        grid=(n // bn,),
        in_specs=[
            pl.BlockSpec((bn, cp), lambda i: (i, 0)),
            pl.BlockSpec((bn, cp), lambda i: (i, 0)),
            pl.BlockSpec((bn, 16), lambda i: (i, 0)),
            pl.BlockSpec((1, cp), lambda i: (0, 0)),
        ],
        out_specs=pl.BlockSpec((bn, cp), lambda i: (i, 0)),
        out_shape=jax.ShapeDtypeStruct((n, cp), jnp.float32),
    )(q0, q1, dinv16, b2p)


def kernel(x, edge_index, W1, b1, W2, b2):
    n, dx = x.shape
    h = W1.shape[1]
    c = W2.shape[1]
    e = edge_index.shape[1]
    cp = _ceil_to(c, 16)
    bn = 400
    assert n % bn == 0 and n % NS == 0

    src = edge_index[0]
    dst = edge_index[1]

    kcd = _ceil_to(_ceil_to(e, NW * CH) // (NW * CH), 2)
    td = NW * kcd * CH
    dstd = jnp.concatenate([dst, jnp.full((td - e,), n, jnp.int32)])
    dstd = dstd.reshape(NW, kcd, CH)
    srcd = jnp.concatenate([
        jnp.zeros((e,), jnp.int32), jnp.ones((td - e,), jnp.int32)
    ]).reshape(NW, kcd, CH)
    ones_tab = jnp.concatenate(
        [jnp.ones((1, 16), jnp.float32), jnp.zeros((7, 16), jnp.float32)])

    e2 = e + n
    kc = _ceil_to(_ceil_to(e2, NW * CH) // (NW * CH), 2)
    t2 = NW * kc * CH
    loop_idx = jnp.arange(n, dtype=jnp.int32)
    src_all = jnp.concatenate(
        [src, loop_idx, jnp.full((t2 - e2,), n, jnp.int32)]).reshape(NW, kc, CH)
    dst_all = jnp.concatenate(
        [dst, loop_idx, jnp.zeros((t2 - e2,), jnp.int32)]).reshape(NW, kc, CH)

    nd = _ceil_to(n + 1, NS * 8)
    na = _ceil_to(n, NS * 8)
    degp = _sc_scatter_rows(srcd, dstd, ones_tab, nd, 16)

    g1, dinv16 = _tc_stage1(x, W1, degp[0, :n], degp[1, :n], bn)
    g1p = jnp.concatenate([g1, jnp.zeros((16, h), jnp.float32)])

    p = _sc_scatter_rows(src_all, dst_all, g1p, na, h)

    w2p = jnp.pad(W2, ((0, 0), (0, cp - c)))
    g2 = _tc_stage2(p[0, :n], p[1, :n], dinv16, b1.reshape(1, h), w2p, bn)
    g2p = jnp.concatenate([g2, jnp.zeros((16, cp), jnp.float32)])

    q = _sc_scatter_rows(src_all, dst_all, g2p, na, cp)

    b2p = jnp.pad(b2, (0, cp - c)).reshape(1, cp)
    out = _tc_stage3(q[0, :n], q[1, :n], dinv16, b2p, bn)

    diag_c = _sc_scatter_rows(src_all, dst_all, g1p, na, h, mode="serial")
    diag_d = _sc_scatter_rows(src_all, dst_all, g1p, na, h, mode="phased")
    return out[:, :c], diag_c[0, 0, :8], diag_d[0, 0, :8]


# speedup vs baseline: 1.2396x; 1.2396x over previous
import functools

import jax
import jax.numpy as jnp
from jax import lax
from jax.experimental import pallas as pl
from jax.experimental.pallas import tpu as pltpu
from jax.experimental.pallas import tpu_sc as plsc

NC = 2
NS = 16
NW = NC * NS
CH = 96


def _ceil_to(a: int, m: int) -> int:
    return ((a + m - 1) // m) * m


def _sc_scatter_rows(src_idx, dst_idx, table, n_out: int, d: int):
    kc = src_idx.shape[1]
    rpt = n_out // NS
    mesh = plsc.VectorSubcoreMesh(core_axis_name="c", subcore_axis_name="s")

    @functools.partial(
        pl.kernel,
        out_type=jax.ShapeDtypeStruct((NC, n_out, d), jnp.float32),
        mesh=mesh,
        scratch_types=[
            pltpu.VMEM((kc, CH), jnp.int32),
            pltpu.VMEM((kc, CH), jnp.int32),
            pltpu.VMEM((CH, d), jnp.float32),
            pltpu.VMEM((CH, d), jnp.float32),
            pltpu.VMEM_SHARED((n_out, d), jnp.float32),
            pltpu.SemaphoreType.DMA,
            pltpu.SemaphoreType.DMA,
        ],
        compiler_params=pltpu.CompilerParams(use_tc_tiling_on_sc=False),
    )
    def k(src_hbm, dst_hbm, tab_hbm, zero_hbm, out_hbm, sv, dv, ra, rb, acc,
          sa, sb):
        c = lax.axis_index("c")
        s = lax.axis_index("s")
        wid = c * NS + s
        pltpu.sync_copy(zero_hbm.at[pl.ds(s * rpt, rpt)],
                        acc.at[pl.ds(s * rpt, rpt)])
        pltpu.sync_copy(src_hbm.at[wid], sv)
        pltpu.sync_copy(dst_hbm.at[wid], dv)
        plsc.subcore_barrier()

        @pl.loop(0, kc, step=2)
        def _(j):
            pltpu.async_copy(tab_hbm.at[sv.at[j]], ra, sa)
            pltpu.async_copy(tab_hbm.at[sv.at[j + 1]], rb, sb)
            pltpu.make_async_copy(tab_hbm.at[sv.at[j]], ra, sa).wait()
            pltpu.make_async_copy(tab_hbm.at[sv.at[j + 1]], rb, sb).wait()
            pltpu.sync_copy(ra, acc.at[dv.at[j]], add=True)
            pltpu.sync_copy(rb, acc.at[dv.at[j + 1]], add=True)

        plsc.subcore_barrier()
        pltpu.sync_copy(acc.at[pl.ds(s * rpt, rpt)],
                        out_hbm.at[c, pl.ds(s * rpt, rpt)])

    zero = jnp.zeros((n_out, d), jnp.float32)
    return k(src_idx, dst_idx, table, zero)


def _tc_stage1(x, w1, deg0, deg1, bn: int):
    n, dx = x.shape
    h = w1.shape[1]

    def body(x_ref, w_ref, d0_ref, d1_ref, g_ref, di_ref):
        deg = d0_ref[:, 0:1] + d1_ref[:, 0:1] + 1.0
        dinv = lax.rsqrt(deg)
        hh = jnp.dot(x_ref[...], w_ref[...], preferred_element_type=jnp.float32)
        g_ref[...] = hh * dinv
        di_ref[...] = jnp.broadcast_to(dinv, di_ref.shape)

    return pl.pallas_call(
        body,
        grid=(n // bn,),
        in_specs=[
            pl.BlockSpec((bn, dx), lambda i: (i, 0)),
            pl.BlockSpec((dx, h), lambda i: (0, 0)),
            pl.BlockSpec((bn, 16), lambda i: (i, 0)),
            pl.BlockSpec((bn, 16), lambda i: (i, 0)),
        ],
        out_specs=[
            pl.BlockSpec((bn, h), lambda i: (i, 0)),
            pl.BlockSpec((bn, 16), lambda i: (i, 0)),
        ],
        out_shape=[
            jax.ShapeDtypeStruct((n, h), jnp.float32),
            jax.ShapeDtypeStruct((n, 16), jnp.float32),
        ],
    )(x, w1, deg0, deg1)


def _tc_stage2(p0, p1, dinv16, b1, w2p, bn: int):
    n, h = p0.shape
    cp = w2p.shape[1]

    def body(p0_ref, p1_ref, di_ref, b_ref, w_ref, g_ref):
        di = di_ref[:, 0:1]
        a = jnp.maximum((p0_ref[...] + p1_ref[...]) * di + b_ref[...], 0.0)
        hh = jnp.dot(a, w_ref[...], preferred_element_type=jnp.float32)
        g_ref[...] = hh * di

    return pl.pallas_call(
        body,
        grid=(n // bn,),
        in_specs=[
            pl.BlockSpec((bn, h), lambda i: (i, 0)),
            pl.BlockSpec((bn, h), lambda i: (i, 0)),
            pl.BlockSpec((bn, 16), lambda i: (i, 0)),
            pl.BlockSpec((1, h), lambda i: (0, 0)),
            pl.BlockSpec((h, cp), lambda i: (0, 0)),
        ],
        out_specs=pl.BlockSpec((bn, cp), lambda i: (i, 0)),
        out_shape=jax.ShapeDtypeStruct((n, cp), jnp.float32),
    )(p0, p1, dinv16, b1, w2p)


def _tc_stage3(q0, q1, dinv16, b2p, bn: int):
    n, cp = q0.shape

    def body(q0_ref, q1_ref, di_ref, b_ref, o_ref):
        di = di_ref[:, 0:1]
        o_ref[...] = (q0_ref[...] + q1_ref[...]) * di + b_ref[...]

    return pl.pallas_call(
        body,
        grid=(n // bn,),
        in_specs=[
            pl.BlockSpec((bn, cp), lambda i: (i, 0)),
            pl.BlockSpec((bn, cp), lambda i: (i, 0)),
            pl.BlockSpec((bn, 16), lambda i: (i, 0)),
            pl.BlockSpec((1, cp), lambda i: (0, 0)),
        ],
        out_specs=pl.BlockSpec((bn, cp), lambda i: (i, 0)),
        out_shape=jax.ShapeDtypeStruct((n, cp), jnp.float32),
    )(q0, q1, dinv16, b2p)


def kernel(x, edge_index, W1, b1, W2, b2):
    n, dx = x.shape
    h = W1.shape[1]
    c = W2.shape[1]
    e = edge_index.shape[1]
    cp = _ceil_to(c, 16)
    bn = 400
    assert n % bn == 0 and n % NS == 0

    src = edge_index[0]
    dst = edge_index[1]

    kcd = _ceil_to(_ceil_to(e, NW * CH) // (NW * CH), 2)
    td = NW * kcd * CH
    dstd = jnp.concatenate([dst, jnp.full((td - e,), n, jnp.int32)])
    dstd = dstd.reshape(NW, kcd, CH)
    srcd = jnp.concatenate([
        jnp.zeros((e,), jnp.int32), jnp.ones((td - e,), jnp.int32)
    ]).reshape(NW, kcd, CH)
    ones_tab = jnp.concatenate(
        [jnp.ones((1, 16), jnp.float32), jnp.zeros((7, 16), jnp.float32)])

    e2 = e + n
    kc = _ceil_to(_ceil_to(e2, NW * CH) // (NW * CH), 2)
    t2 = NW * kc * CH
    loop_idx = jnp.arange(n, dtype=jnp.int32)
    src_all = jnp.concatenate(
        [src, loop_idx, jnp.full((t2 - e2,), n, jnp.int32)]).reshape(NW, kc, CH)
    dst_all = jnp.concatenate(
        [dst, loop_idx, jnp.zeros((t2 - e2,), jnp.int32)]).reshape(NW, kc, CH)

    nd = _ceil_to(n + 1, NS * 8)
    na = _ceil_to(n, NS * 8)
    degp = _sc_scatter_rows(srcd, dstd, ones_tab, nd, 16)

    g1, dinv16 = _tc_stage1(x, W1, degp[0, :n], degp[1, :n], bn)
    g1p = jnp.concatenate([g1, jnp.zeros((16, h), jnp.float32)])

    p = _sc_scatter_rows(src_all, dst_all, g1p, na, h)

    w2p = jnp.pad(W2, ((0, 0), (0, cp - c)))
    g2 = _tc_stage2(p[0, :n], p[1, :n], dinv16, b1.reshape(1, h), w2p, bn)
    g2p = jnp.concatenate([g2, jnp.zeros((16, cp), jnp.float32)])

    q = _sc_scatter_rows(src_all, dst_all, g2p, na, cp)

    b2p = jnp.pad(b2, (0, cp - c)).reshape(1, cp)
    out = _tc_stage3(q[0, :n], q[1, :n], dinv16, b2p, bn)

    return out[:, :c]


# speedup vs baseline: 4.8100x; 3.8803x over previous
import functools

import jax
import jax.numpy as jnp
from jax import lax
from jax.experimental import pallas as pl
from jax.experimental.pallas import tpu as pltpu
from jax.experimental.pallas import tpu_sc as plsc

NC = 2
NS = 16
NW = NC * NS
CH = 96


def _ceil_to(a: int, m: int) -> int:
    return ((a + m - 1) // m) * m


def _sc_scatter_rows(src_idx, dst_idx, table, n_out: int, d: int):
    kc = src_idx.shape[1]
    rpt = n_out // NS
    mesh = plsc.VectorSubcoreMesh(core_axis_name="c", subcore_axis_name="s")

    @functools.partial(
        pl.kernel,
        out_type=jax.ShapeDtypeStruct((NC, n_out, d), jnp.float32),
        mesh=mesh,
        scratch_types=[
            pltpu.VMEM((kc, CH), jnp.int32),
            pltpu.VMEM((kc, CH), jnp.int32),
            pltpu.VMEM((CH, d), jnp.float32),
            pltpu.VMEM((CH, d), jnp.float32),
            pltpu.VMEM_SHARED((n_out, d), jnp.float32),
            pltpu.SemaphoreType.DMA,
            pltpu.SemaphoreType.DMA,
        ],
        compiler_params=pltpu.CompilerParams(use_tc_tiling_on_sc=False),
    )
    def k(src_hbm, dst_hbm, tab_hbm, zero_hbm, out_hbm, sv, dv, ra, rb, acc,
          sa, sb):
        c = lax.axis_index("c")
        s = lax.axis_index("s")
        wid = c * NS + s
        pltpu.sync_copy(zero_hbm.at[pl.ds(s * rpt, rpt)],
                        acc.at[pl.ds(s * rpt, rpt)])
        pltpu.sync_copy(src_hbm.at[wid], sv)
        pltpu.sync_copy(dst_hbm.at[wid], dv)
        plsc.subcore_barrier()

        @pl.loop(0, kc, step=2)
        def _(j):
            pltpu.async_copy(tab_hbm.at[sv.at[j]], ra, sa)
            pltpu.async_copy(tab_hbm.at[sv.at[j + 1]], rb, sb)
            pltpu.make_async_copy(tab_hbm.at[sv.at[j]], ra, sa).wait()
            pltpu.make_async_copy(tab_hbm.at[sv.at[j + 1]], rb, sb).wait()
            pltpu.sync_copy(ra, acc.at[dv.at[j]], add=True)
            pltpu.sync_copy(rb, acc.at[dv.at[j + 1]], add=True)

        plsc.subcore_barrier()
        pltpu.sync_copy(acc.at[pl.ds(s * rpt, rpt)],
                        out_hbm.at[c, pl.ds(s * rpt, rpt)])

    zero = jnp.zeros((n_out, d), jnp.float32)
    return k(src_idx, dst_idx, table, zero)


def _sc_count_rows(dst_idx, n_out: int):
    d = 16
    kc = dst_idx.shape[1]
    rpt = n_out // NS
    mesh = plsc.VectorSubcoreMesh(core_axis_name="c", subcore_axis_name="s")

    @functools.partial(
        pl.kernel,
        out_type=jax.ShapeDtypeStruct((NC, n_out, d), jnp.float32),
        mesh=mesh,
        scratch_types=[
            pltpu.VMEM((kc, CH), jnp.int32),
            pltpu.VMEM((CH, d), jnp.float32),
            pltpu.VMEM_SHARED((n_out, d), jnp.float32),
        ],
        compiler_params=pltpu.CompilerParams(use_tc_tiling_on_sc=False),
    )
    def k(dst_hbm, ones_hbm, zero_hbm, out_hbm, dv, ra, acc):
        c = lax.axis_index("c")
        s = lax.axis_index("s")
        wid = c * NS + s
        pltpu.sync_copy(zero_hbm.at[pl.ds(s * rpt, rpt)],
                        acc.at[pl.ds(s * rpt, rpt)])
        pltpu.sync_copy(dst_hbm.at[wid], dv)
        pltpu.sync_copy(ones_hbm, ra)
        plsc.subcore_barrier()

        @pl.loop(0, kc)
        def _(j):
            pltpu.sync_copy(ra, acc.at[dv.at[j]], add=True)

        plsc.subcore_barrier()
        pltpu.sync_copy(acc.at[pl.ds(s * rpt, rpt)],
                        out_hbm.at[c, pl.ds(s * rpt, rpt)])

    zero = jnp.zeros((n_out, d), jnp.float32)
    ones = jnp.ones((CH, d), jnp.float32)
    return k(dst_idx, ones, zero)


def _tc_stage1(x, w1, deg0, deg1, bn: int):
    n, dx = x.shape
    h = w1.shape[1]

    def body(x_ref, w_ref, d0_ref, d1_ref, g_ref, di_ref):
        deg = d0_ref[:, 0:1] + d1_ref[:, 0:1] + 1.0
        dinv = lax.rsqrt(deg)
        hh = jnp.dot(x_ref[...], w_ref[...], preferred_element_type=jnp.float32)
        g_ref[...] = hh * dinv
        di_ref[...] = jnp.broadcast_to(dinv, di_ref.shape)

    return pl.pallas_call(
        body,
        grid=(n // bn,),
        in_specs=[
            pl.BlockSpec((bn, dx), lambda i: (i, 0)),
            pl.BlockSpec((dx, h), lambda i: (0, 0)),
            pl.BlockSpec((bn, 16), lambda i: (i, 0)),
            pl.BlockSpec((bn, 16), lambda i: (i, 0)),
        ],
        out_specs=[
            pl.BlockSpec((bn, h), lambda i: (i, 0)),
            pl.BlockSpec((bn, 16), lambda i: (i, 0)),
        ],
        out_shape=[
            jax.ShapeDtypeStruct((n, h), jnp.float32),
            jax.ShapeDtypeStruct((n, 16), jnp.float32),
        ],
    )(x, w1, deg0, deg1)


def _tc_stage2(p0, p1, dinv16, b1, w2p, bn: int):
    n, h = p0.shape
    cp = w2p.shape[1]

    def body(p0_ref, p1_ref, di_ref, b_ref, w_ref, g_ref):
        di = di_ref[:, 0:1]
        a = jnp.maximum((p0_ref[...] + p1_ref[...]) * di + b_ref[...], 0.0)
        hh = jnp.dot(a, w_ref[...], preferred_element_type=jnp.float32)
        g_ref[...] = hh * di

    return pl.pallas_call(
        body,
        grid=(n // bn,),
        in_specs=[
            pl.BlockSpec((bn, h), lambda i: (i, 0)),
            pl.BlockSpec((bn, h), lambda i: (i, 0)),
            pl.BlockSpec((bn, 16), lambda i: (i, 0)),
            pl.BlockSpec((1, h), lambda i: (0, 0)),
            pl.BlockSpec((h, cp), lambda i: (0, 0)),
        ],
        out_specs=pl.BlockSpec((bn, cp), lambda i: (i, 0)),
        out_shape=jax.ShapeDtypeStruct((n, cp), jnp.float32),
    )(p0, p1, dinv16, b1, w2p)


def _tc_stage3(q0, q1, dinv16, b2p, bn: int):
    n, cp = q0.shape

    def body(q0_ref, q1_ref, di_ref, b_ref, o_ref):
        di = di_ref[:, 0:1]
        o_ref[...] = (q0_ref[...] + q1_ref[...]) * di + b_ref[...]

    return pl.pallas_call(
        body,
        grid=(n // bn,),
        in_specs=[
            pl.BlockSpec((bn, cp), lambda i: (i, 0)),
            pl.BlockSpec((bn, cp), lambda i: (i, 0)),
            pl.BlockSpec((bn, 16), lambda i: (i, 0)),
            pl.BlockSpec((1, cp), lambda i: (0, 0)),
        ],
        out_specs=pl.BlockSpec((bn, cp), lambda i: (i, 0)),
        out_shape=jax.ShapeDtypeStruct((n, cp), jnp.float32),
    )(q0, q1, dinv16, b2p)


def kernel(x, edge_index, W1, b1, W2, b2):
    n, dx = x.shape
    h = W1.shape[1]
    c = W2.shape[1]
    e = edge_index.shape[1]
    cp = _ceil_to(c, 16)
    bn = 400
    assert n % bn == 0 and n % NS == 0

    src = edge_index[0]
    dst = edge_index[1]

    kcd = _ceil_to(_ceil_to(e, NW * CH) // (NW * CH), 2)
    td = NW * kcd * CH
    dstd = jnp.concatenate([dst, jnp.full((td - e,), n, jnp.int32)])
    dstd = dstd.reshape(NW, kcd, CH)

    e2 = e + n
    kc = _ceil_to(_ceil_to(e2, NW * CH) // (NW * CH), 2)
    t2 = NW * kc * CH
    loop_idx = jnp.arange(n, dtype=jnp.int32)
    src_all = jnp.concatenate(
        [src, loop_idx, jnp.full((t2 - e2,), n, jnp.int32)]).reshape(NW, kc, CH)
    dst_all = jnp.concatenate(
        [dst, loop_idx, jnp.zeros((t2 - e2,), jnp.int32)]).reshape(NW, kc, CH)

    nd = _ceil_to(n + 1, NS * 8)
    na = _ceil_to(n, NS * 8)
    degp = _sc_count_rows(dstd, nd)

    g1, dinv16 = _tc_stage1(x, W1, degp[0, :n], degp[1, :n], bn)
    g1p = jnp.concatenate([g1, jnp.zeros((16, h), jnp.float32)])

    p = _sc_scatter_rows(src_all, dst_all, g1p, na, h)

    w2p = jnp.pad(W2, ((0, 0), (0, cp - c)))
    g2 = _tc_stage2(p[0, :n], p[1, :n], dinv16, b1.reshape(1, h), w2p, bn)
    g2p = jnp.concatenate([g2, jnp.zeros((16, cp), jnp.float32)])

    q = _sc_scatter_rows(src_all, dst_all, g2p, na, cp)

    b2p = jnp.pad(b2, (0, cp - c)).reshape(1, cp)
    out = _tc_stage3(q[0, :n], q[1, :n], dinv16, b2p, bn)

    return out[:, :c]


# speedup vs baseline: 6.1577x; 1.2802x over previous
import functools

import jax
import jax.numpy as jnp
from jax import lax
from jax.experimental import pallas as pl
from jax.experimental.pallas import tpu as pltpu
from jax.experimental.pallas import tpu_sc as plsc

NC = 2
NS = 16
NW = NC * NS
CH = 96


def _ceil_to(a: int, m: int) -> int:
    return ((a + m - 1) // m) * m


def _sc_scatter_rows(src_idx, dst_idx, table, n_out: int, d: int):
    kc = src_idx.shape[1]
    rpt = n_out // NS
    mesh = plsc.VectorSubcoreMesh(core_axis_name="c", subcore_axis_name="s")

    @functools.partial(
        pl.kernel,
        out_type=jax.ShapeDtypeStruct((NC, n_out, d), jnp.float32),
        mesh=mesh,
        scratch_types=[
            pltpu.VMEM((kc, CH), jnp.int32),
            pltpu.VMEM((kc, CH), jnp.int32),
            pltpu.VMEM((CH, d), jnp.float32),
            pltpu.VMEM((CH, d), jnp.float32),
            pltpu.VMEM_SHARED((n_out, d), jnp.float32),
            pltpu.SemaphoreType.DMA,
            pltpu.SemaphoreType.DMA,
        ],
        compiler_params=pltpu.CompilerParams(use_tc_tiling_on_sc=False),
    )
    def k(src_hbm, dst_hbm, tab_hbm, zero_hbm, out_hbm, sv, dv, ra, rb, acc,
          sa, sb):
        c = lax.axis_index("c")
        s = lax.axis_index("s")
        wid = c * NS + s
        pltpu.sync_copy(zero_hbm.at[pl.ds(s * rpt, rpt)],
                        acc.at[pl.ds(s * rpt, rpt)])
        pltpu.sync_copy(src_hbm.at[wid], sv)
        pltpu.sync_copy(dst_hbm.at[wid], dv)
        plsc.subcore_barrier()

        @pl.loop(0, kc, step=2)
        def _(j):
            pltpu.async_copy(tab_hbm.at[sv.at[j]], ra, sa)
            pltpu.async_copy(tab_hbm.at[sv.at[j + 1]], rb, sb)
            pltpu.make_async_copy(tab_hbm.at[sv.at[j]], ra, sa).wait()
            pltpu.make_async_copy(tab_hbm.at[sv.at[j + 1]], rb, sb).wait()
            pltpu.sync_copy(ra, acc.at[dv.at[j]], add=True)
            pltpu.sync_copy(rb, acc.at[dv.at[j + 1]], add=True)

        plsc.subcore_barrier()
        pltpu.sync_copy(acc.at[pl.ds(s * rpt, rpt)],
                        out_hbm.at[c, pl.ds(s * rpt, rpt)])

    zero = jnp.zeros((n_out, d), jnp.float32)
    return k(src_idx, dst_idx, table, zero)


def _sc_count_rows(dst_idx, n_out: int):
    d = 16
    kc = dst_idx.shape[1]
    rpt = n_out // NS
    mesh = plsc.VectorSubcoreMesh(core_axis_name="c", subcore_axis_name="s")

    @functools.partial(
        pl.kernel,
        out_type=jax.ShapeDtypeStruct((NC, n_out, d), jnp.float32),
        mesh=mesh,
        scratch_types=[
            pltpu.VMEM((kc, CH), jnp.int32),
            pltpu.VMEM((CH, d), jnp.float32),
            pltpu.VMEM_SHARED((n_out, d), jnp.float32),
        ],
        compiler_params=pltpu.CompilerParams(use_tc_tiling_on_sc=False),
    )
    def k(dst_hbm, ones_hbm, zero_hbm, out_hbm, dv, ra, acc):
        c = lax.axis_index("c")
        s = lax.axis_index("s")
        wid = c * NS + s
        pltpu.sync_copy(zero_hbm.at[pl.ds(s * rpt, rpt)],
                        acc.at[pl.ds(s * rpt, rpt)])
        pltpu.sync_copy(dst_hbm.at[wid], dv)
        pltpu.sync_copy(ones_hbm, ra)
        plsc.subcore_barrier()

        @pl.loop(0, kc)
        def _(j):
            pltpu.sync_copy(ra, acc.at[dv.at[j]], add=True)

        plsc.subcore_barrier()
        pltpu.sync_copy(acc.at[pl.ds(s * rpt, rpt)],
                        out_hbm.at[c, pl.ds(s * rpt, rpt)])

    zero = jnp.zeros((n_out, d), jnp.float32)
    ones = jnp.ones((CH, d), jnp.float32)
    return k(dst_idx, ones, zero)


def _tc_stage1(x, w1, deg0, deg1, bn: int):
    n, dx = x.shape
    h = w1.shape[1]

    def body(x_ref, w_ref, d0_ref, d1_ref, g_ref, di_ref):
        deg = d0_ref[:, 0:1] + d1_ref[:, 0:1] + 1.0
        dinv = lax.rsqrt(deg)
        hh = jnp.dot(x_ref[...], w_ref[...], preferred_element_type=jnp.float32)
        g_ref[...] = hh * dinv
        di_ref[...] = jnp.broadcast_to(dinv, di_ref.shape)

    return pl.pallas_call(
        body,
        grid=(n // bn,),
        in_specs=[
            pl.BlockSpec((bn, dx), lambda i: (i, 0)),
            pl.BlockSpec((dx, h), lambda i: (0, 0)),
            pl.BlockSpec((bn, 16), lambda i: (i, 0)),
            pl.BlockSpec((bn, 16), lambda i: (i, 0)),
        ],
        out_specs=[
            pl.BlockSpec((bn, h), lambda i: (i, 0)),
            pl.BlockSpec((bn, 16), lambda i: (i, 0)),
        ],
        out_shape=[
            jax.ShapeDtypeStruct((n, h), jnp.float32),
            jax.ShapeDtypeStruct((n, 16), jnp.float32),
        ],
    )(x, w1, deg0, deg1)


def _tc_stage2(p0, p1, g1, dinv16, b1, w2p, bn: int):
    n, h = p0.shape
    cp = w2p.shape[1]

    def body(p0_ref, p1_ref, g1_ref, di_ref, b_ref, w_ref, g_ref):
        di = di_ref[:, 0:1]
        s = p0_ref[...] + p1_ref[...] + g1_ref[...]
        a = jnp.maximum(s * di + b_ref[...], 0.0)
        hh = jnp.dot(a, w_ref[...], preferred_element_type=jnp.float32)
        g_ref[...] = hh * di

    return pl.pallas_call(
        body,
        grid=(n // bn,),
        in_specs=[
            pl.BlockSpec((bn, h), lambda i: (i, 0)),
            pl.BlockSpec((bn, h), lambda i: (i, 0)),
            pl.BlockSpec((bn, h), lambda i: (i, 0)),
            pl.BlockSpec((bn, 16), lambda i: (i, 0)),
            pl.BlockSpec((1, h), lambda i: (0, 0)),
            pl.BlockSpec((h, cp), lambda i: (0, 0)),
        ],
        out_specs=pl.BlockSpec((bn, cp), lambda i: (i, 0)),
        out_shape=jax.ShapeDtypeStruct((n, cp), jnp.float32),
    )(p0, p1, g1, dinv16, b1, w2p)


def _tc_stage3(q0, q1, g2, dinv16, b2p, bn: int):
    n, cp = q0.shape

    def body(q0_ref, q1_ref, g2_ref, di_ref, b_ref, o_ref):
        di = di_ref[:, 0:1]
        s = q0_ref[...] + q1_ref[...] + g2_ref[...]
        o_ref[...] = s * di + b_ref[...]

    return pl.pallas_call(
        body,
        grid=(n // bn,),
        in_specs=[
            pl.BlockSpec((bn, cp), lambda i: (i, 0)),
            pl.BlockSpec((bn, cp), lambda i: (i, 0)),
            pl.BlockSpec((bn, cp), lambda i: (i, 0)),
            pl.BlockSpec((bn, 16), lambda i: (i, 0)),
            pl.BlockSpec((1, cp), lambda i: (0, 0)),
        ],
        out_specs=pl.BlockSpec((bn, cp), lambda i: (i, 0)),
        out_shape=jax.ShapeDtypeStruct((n, cp), jnp.float32),
    )(q0, q1, g2, dinv16, b2p)


def kernel(x, edge_index, W1, b1, W2, b2):
    n, dx = x.shape
    h = W1.shape[1]
    c = W2.shape[1]
    e = edge_index.shape[1]
    cp = _ceil_to(c, 16)
    bn = 400
    assert n % bn == 0 and n % NS == 0

    src = edge_index[0]
    dst = edge_index[1]

    kc = _ceil_to(_ceil_to(e, NW * CH) // (NW * CH), 2)
    t2 = NW * kc * CH
    pad = t2 - e
    src_all = jnp.concatenate(
        [src, jnp.arange(pad, dtype=jnp.int32) % n]).reshape(NW, kc, CH)
    dst_all = jnp.concatenate(
        [dst, jnp.full((pad,), n, jnp.int32)]).reshape(NW, kc, CH)

    na = _ceil_to(n + 1, NS * 8)
    degp = _sc_count_rows(dst_all, na)

    g1, dinv16 = _tc_stage1(x, W1, degp[0, :n], degp[1, :n], bn)
    g1p = jnp.concatenate([g1, jnp.zeros((na - n, h), jnp.float32)])

    p = _sc_scatter_rows(src_all, dst_all, g1p, na, h)

    w2p = jnp.pad(W2, ((0, 0), (0, cp - c)))
    g2 = _tc_stage2(p[0, :n], p[1, :n], g1, dinv16, b1.reshape(1, h), w2p, bn)
    g2p = jnp.concatenate([g2, jnp.zeros((na - n, cp), jnp.float32)])

    q = _sc_scatter_rows(src_all, dst_all, g2p, na, cp)

    b2p = jnp.pad(b2, (0, cp - c)).reshape(1, cp)
    out = _tc_stage3(q[0, :n], q[1, :n], g2, dinv16, b2p, bn)

    return out[:, :c]


# speedup vs baseline: 6.6826x; 1.0852x over previous
import functools

import jax
import jax.numpy as jnp
from jax import lax
from jax.experimental import pallas as pl
from jax.experimental.pallas import tpu as pltpu
from jax.experimental.pallas import tpu_sc as plsc

NC = 2
NS = 16
NW = NC * NS
CH = 112


def _ceil_to(a: int, m: int) -> int:
    return ((a + m - 1) // m) * m


def _sc_scatter_rows(src_idx, dst_idx, table, n_out: int, d: int):
    kc = src_idx.shape[1]
    rpt = n_out // NS
    mesh = plsc.VectorSubcoreMesh(core_axis_name="c", subcore_axis_name="s")

    @functools.partial(
        pl.kernel,
        out_type=jax.ShapeDtypeStruct((NC, n_out, d), jnp.float32),
        mesh=mesh,
        scratch_types=[
            pltpu.VMEM((kc, CH), jnp.int32),
            pltpu.VMEM((kc, CH), jnp.int32),
            pltpu.VMEM((CH, d), jnp.float32),
            pltpu.VMEM((CH, d), jnp.float32),
            pltpu.VMEM_SHARED((n_out, d), jnp.float32),
            pltpu.SemaphoreType.DMA,
            pltpu.SemaphoreType.DMA,
        ],
        compiler_params=pltpu.CompilerParams(use_tc_tiling_on_sc=False),
    )
    def k(src_hbm, dst_hbm, tab_hbm, zero_hbm, out_hbm, sv, dv, ra, rb, acc,
          sa, sb):
        c = lax.axis_index("c")
        s = lax.axis_index("s")
        wid = c * NS + s
        pltpu.sync_copy(zero_hbm.at[pl.ds(s * rpt, rpt)],
                        acc.at[pl.ds(s * rpt, rpt)])
        pltpu.sync_copy(src_hbm.at[wid], sv)
        pltpu.sync_copy(dst_hbm.at[wid], dv)
        plsc.subcore_barrier()

        @pl.loop(0, kc, step=2)
        def _(j):
            pltpu.async_copy(tab_hbm.at[sv.at[j]], ra, sa)
            pltpu.async_copy(tab_hbm.at[sv.at[j + 1]], rb, sb)
            pltpu.make_async_copy(tab_hbm.at[sv.at[j]], ra, sa).wait()
            pltpu.make_async_copy(tab_hbm.at[sv.at[j + 1]], rb, sb).wait()
            pltpu.sync_copy(ra, acc.at[dv.at[j]], add=True)
            pltpu.sync_copy(rb, acc.at[dv.at[j + 1]], add=True)

        plsc.subcore_barrier()
        pltpu.sync_copy(acc.at[pl.ds(s * rpt, rpt)],
                        out_hbm.at[c, pl.ds(s * rpt, rpt)])

    zero = jnp.zeros((n_out, d), jnp.float32)
    return k(src_idx, dst_idx, table, zero)


def _sc_count_rows(dst_idx, n_out: int):
    d = 16
    kc = dst_idx.shape[1]
    rpt = n_out // NS
    mesh = plsc.VectorSubcoreMesh(core_axis_name="c", subcore_axis_name="s")

    @functools.partial(
        pl.kernel,
        out_type=jax.ShapeDtypeStruct((NC, n_out, d), jnp.float32),
        mesh=mesh,
        scratch_types=[
            pltpu.VMEM((kc, CH), jnp.int32),
            pltpu.VMEM((CH, d), jnp.float32),
            pltpu.VMEM_SHARED((n_out, d), jnp.float32),
        ],
        compiler_params=pltpu.CompilerParams(use_tc_tiling_on_sc=False),
    )
    def k(dst_hbm, ones_hbm, zero_hbm, out_hbm, dv, ra, acc):
        c = lax.axis_index("c")
        s = lax.axis_index("s")
        wid = c * NS + s
        pltpu.sync_copy(zero_hbm.at[pl.ds(s * rpt, rpt)],
                        acc.at[pl.ds(s * rpt, rpt)])
        pltpu.sync_copy(dst_hbm.at[wid], dv)
        pltpu.sync_copy(ones_hbm, ra)
        plsc.subcore_barrier()

        @pl.loop(0, kc)
        def _(j):
            pltpu.sync_copy(ra, acc.at[dv.at[j]], add=True)

        plsc.subcore_barrier()
        pltpu.sync_copy(acc.at[pl.ds(s * rpt, rpt)],
                        out_hbm.at[c, pl.ds(s * rpt, rpt)])

    zero = jnp.zeros((n_out, d), jnp.float32)
    ones = jnp.ones((CH, d), jnp.float32)
    return k(dst_idx, ones, zero)


def _tc_stage1(x, w1, degp, bn: int):
    n, dx = x.shape
    h = w1.shape[1]

    def body(x_ref, w_ref, d0_ref, d1_ref, g_ref, di_ref):
        deg = d0_ref[0, :, 0:1] + d1_ref[0, :, 0:1] + 1.0
        dinv = lax.rsqrt(deg)
        hh = jnp.dot(x_ref[...], w_ref[...], preferred_element_type=jnp.float32)
        g_ref[...] = hh * dinv
        di_ref[...] = jnp.broadcast_to(dinv, di_ref.shape)

    return pl.pallas_call(
        body,
        grid=(n // bn,),
        in_specs=[
            pl.BlockSpec((bn, dx), lambda i: (i, 0)),
            pl.BlockSpec((dx, h), lambda i: (0, 0)),
            pl.BlockSpec((1, bn, 16), lambda i: (0, i, 0)),
            pl.BlockSpec((1, bn, 16), lambda i: (1, i, 0)),
        ],
        out_specs=[
            pl.BlockSpec((bn, h), lambda i: (i, 0)),
            pl.BlockSpec((bn, 16), lambda i: (i, 0)),
        ],
        out_shape=[
            jax.ShapeDtypeStruct((n, h), jnp.float32),
            jax.ShapeDtypeStruct((n, 16), jnp.float32),
        ],
    )(x, w1, degp, degp)


def _tc_stage2(p0, p1, g1, dinv16, b1, w2p, bn: int):
    n, h = g1.shape
    cp = w2p.shape[1]

    def body(p0_ref, p1_ref, g1_ref, di_ref, b_ref, w_ref, g_ref):
        di = di_ref[:, 0:1]
        s = p0_ref[0] + p1_ref[0] + g1_ref[...]
        a = jnp.maximum(s * di + b_ref[...], 0.0)
        hh = jnp.dot(a, w_ref[...], preferred_element_type=jnp.float32)
        g_ref[...] = hh * di

    return pl.pallas_call(
        body,
        grid=(n // bn,),
        in_specs=[
            pl.BlockSpec((1, bn, h), lambda i: (0, i, 0)),
            pl.BlockSpec((1, bn, h), lambda i: (1, i, 0)),
            pl.BlockSpec((bn, h), lambda i: (i, 0)),
            pl.BlockSpec((bn, 16), lambda i: (i, 0)),
            pl.BlockSpec((1, h), lambda i: (0, 0)),
            pl.BlockSpec((h, cp), lambda i: (0, 0)),
        ],
        out_specs=pl.BlockSpec((bn, cp), lambda i: (i, 0)),
        out_shape=jax.ShapeDtypeStruct((n, cp), jnp.float32),
    )(p0, p1, g1, dinv16, b1, w2p)


def _tc_stage3(q0, q1, g2, dinv16, b2p, bn: int):
    n, cp = g2.shape

    def body(q0_ref, q1_ref, g2_ref, di_ref, b_ref, o_ref):
        di = di_ref[:, 0:1]
        s = q0_ref[0] + q1_ref[0] + g2_ref[...]
        o_ref[...] = s * di + b_ref[...]

    return pl.pallas_call(
        body,
        grid=(n // bn,),
        in_specs=[
            pl.BlockSpec((1, bn, cp), lambda i: (0, i, 0)),
            pl.BlockSpec((1, bn, cp), lambda i: (1, i, 0)),
            pl.BlockSpec((bn, cp), lambda i: (i, 0)),
            pl.BlockSpec((bn, 16), lambda i: (i, 0)),
            pl.BlockSpec((1, cp), lambda i: (0, 0)),
        ],
        out_specs=pl.BlockSpec((bn, cp), lambda i: (i, 0)),
        out_shape=jax.ShapeDtypeStruct((n, cp), jnp.float32),
    )(q0, q1, g2, dinv16, b2p)


def kernel(x, edge_index, W1, b1, W2, b2):
    n, dx = x.shape
    h = W1.shape[1]
    c = W2.shape[1]
    e = edge_index.shape[1]
    cp = _ceil_to(c, 16)
    bn = 400
    assert n % bn == 0 and n % NS == 0

    src = edge_index[0]
    dst = edge_index[1]

    kc = _ceil_to(_ceil_to(e, NW * CH) // (NW * CH), 2)
    t2 = NW * kc * CH
    pad = t2 - e
    src_all = jnp.concatenate(
        [src, jnp.arange(pad, dtype=jnp.int32) % n]).reshape(NW, kc, CH)
    dst_all = jnp.concatenate(
        [dst, jnp.full((pad,), n, jnp.int32)]).reshape(NW, kc, CH)

    na = _ceil_to(n + 1, NS * 8)
    degp = _sc_count_rows(dst_all, na)

    g1, dinv16 = _tc_stage1(x, W1, degp, bn)
    g1p = jnp.concatenate([g1, jnp.zeros((na - n, h), jnp.float32)])

    p = _sc_scatter_rows(src_all, dst_all, g1p, na, h)

    w2p = jnp.pad(W2, ((0, 0), (0, cp - c)))
    g2 = _tc_stage2(p, p, g1, dinv16, b1.reshape(1, h), w2p, bn)
    g2p = jnp.concatenate([g2, jnp.zeros((na - n, cp), jnp.float32)])

    q = _sc_scatter_rows(src_all, dst_all, g2p, na, cp)

    b2p = jnp.pad(b2, (0, cp - c)).reshape(1, cp)
    out = _tc_stage3(q, q, g2, dinv16, b2p, bn)

    return out[:, :c]


# speedup vs baseline: 7.2118x; 1.0792x over previous
import functools

import jax
import jax.numpy as jnp
from jax import lax
from jax.experimental import pallas as pl
from jax.experimental.pallas import tpu as pltpu
from jax.experimental.pallas import tpu_sc as plsc

NC = 2
NS = 16
NW = NC * NS
CH = 112


def _ceil_to(a: int, m: int) -> int:
    return ((a + m - 1) // m) * m


def _sc_scatter_rows(src_idx, dst_idx, table, n_out: int, d: int):
    kc = src_idx.shape[1]
    rpt = n_out // NS
    mesh = plsc.VectorSubcoreMesh(core_axis_name="c", subcore_axis_name="s")

    @functools.partial(
        pl.kernel,
        out_type=jax.ShapeDtypeStruct((NC, n_out, d), jnp.float32),
        mesh=mesh,
        scratch_types=[
            pltpu.VMEM((kc, CH), jnp.int32),
            pltpu.VMEM((kc, CH), jnp.int32),
            pltpu.VMEM((CH, d), jnp.float32),
            pltpu.VMEM((CH, d), jnp.float32),
            pltpu.VMEM_SHARED((n_out, d), jnp.float32),
            pltpu.SemaphoreType.DMA,
            pltpu.SemaphoreType.DMA,
        ],
        compiler_params=pltpu.CompilerParams(use_tc_tiling_on_sc=False),
    )
    def k(src_hbm, dst_hbm, tab_hbm, zero_hbm, out_hbm, sv, dv, ra, rb, acc,
          sa, sb):
        c = lax.axis_index("c")
        s = lax.axis_index("s")
        wid = c * NS + s
        pltpu.sync_copy(zero_hbm.at[pl.ds(s * rpt, rpt)],
                        acc.at[pl.ds(s * rpt, rpt)])
        pltpu.sync_copy(src_hbm.at[wid], sv)
        pltpu.sync_copy(dst_hbm.at[wid], dv)
        plsc.subcore_barrier()

        @pl.loop(0, kc, step=2)
        def _(j):
            pltpu.async_copy(tab_hbm.at[sv.at[j]], ra, sa)
            pltpu.async_copy(tab_hbm.at[sv.at[j + 1]], rb, sb)
            pltpu.make_async_copy(tab_hbm.at[sv.at[j]], ra, sa).wait()
            pltpu.make_async_copy(tab_hbm.at[sv.at[j + 1]], rb, sb).wait()
            pltpu.sync_copy(ra, acc.at[dv.at[j]], add=True)
            pltpu.sync_copy(rb, acc.at[dv.at[j + 1]], add=True)

        plsc.subcore_barrier()
        pltpu.sync_copy(acc.at[pl.ds(s * rpt, rpt)],
                        out_hbm.at[c, pl.ds(s * rpt, rpt)])

    zero = jnp.zeros((n_out, d), jnp.float32)
    return k(src_idx, dst_idx, table, zero)


def _sc_count_rows(dst_idx, n_out: int):
    d = 16
    kc = dst_idx.shape[1]
    rpt = n_out // NS
    mesh = plsc.VectorSubcoreMesh(core_axis_name="c", subcore_axis_name="s")

    @functools.partial(
        pl.kernel,
        out_type=jax.ShapeDtypeStruct((NC, n_out, d), jnp.float32),
        mesh=mesh,
        scratch_types=[
            pltpu.VMEM((kc, CH), jnp.int32),
            pltpu.VMEM((CH, d), jnp.float32),
            pltpu.VMEM_SHARED((n_out, d), jnp.float32),
        ],
        compiler_params=pltpu.CompilerParams(use_tc_tiling_on_sc=False),
    )
    def k(dst_hbm, ones_hbm, zero_hbm, out_hbm, dv, ra, acc):
        c = lax.axis_index("c")
        s = lax.axis_index("s")
        wid = c * NS + s
        pltpu.sync_copy(zero_hbm.at[pl.ds(s * rpt, rpt)],
                        acc.at[pl.ds(s * rpt, rpt)])
        pltpu.sync_copy(dst_hbm.at[wid], dv)
        pltpu.sync_copy(ones_hbm, ra)
        plsc.subcore_barrier()

        @pl.loop(0, kc)
        def _(j):
            pltpu.sync_copy(ra, acc.at[dv.at[j]], add=True)

        plsc.subcore_barrier()
        pltpu.sync_copy(acc.at[pl.ds(s * rpt, rpt)],
                        out_hbm.at[c, pl.ds(s * rpt, rpt)])

    zero = jnp.zeros((n_out, d), jnp.float32)
    ones = jnp.ones((CH, d), jnp.float32)
    return k(dst_idx, ones, zero)


def _tc_stage1(x, w1, degp, bn: int):
    n, dx = x.shape
    h = w1.shape[1]

    def body(x_ref, w_ref, d0_ref, d1_ref, g_ref, di_ref):
        deg = d0_ref[0, :, 0:1] + d1_ref[0, :, 0:1] + 1.0
        dinv = lax.rsqrt(deg)
        hh = jnp.dot(x_ref[...], w_ref[...], preferred_element_type=jnp.float32)
        g_ref[...] = hh * dinv
        di_ref[...] = jnp.broadcast_to(dinv, di_ref.shape)

    return pl.pallas_call(
        body,
        grid=(n // bn,),
        in_specs=[
            pl.BlockSpec((bn, dx), lambda i: (i, 0)),
            pl.BlockSpec((dx, h), lambda i: (0, 0)),
            pl.BlockSpec((1, bn, 16), lambda i: (0, i, 0)),
            pl.BlockSpec((1, bn, 16), lambda i: (1, i, 0)),
        ],
        out_specs=[
            pl.BlockSpec((bn, h), lambda i: (i, 0)),
            pl.BlockSpec((bn, 16), lambda i: (i, 0)),
        ],
        out_shape=[
            jax.ShapeDtypeStruct((n, h), jnp.float32),
            jax.ShapeDtypeStruct((n, 16), jnp.float32),
        ],
    )(x, w1, degp, degp)


def _tc_stage2(p0, p1, g1, dinv16, b1, w2p, bn: int):
    n, h = g1.shape
    cp = w2p.shape[1]

    def body(p0_ref, p1_ref, g1_ref, di_ref, b_ref, w_ref, g_ref):
        di = di_ref[:, 0:1]
        s = p0_ref[0] + p1_ref[0] + g1_ref[...]
        a = jnp.maximum(s * di + b_ref[...], 0.0)
        hh = jnp.dot(a, w_ref[...], preferred_element_type=jnp.float32)
        g_ref[...] = hh * di

    return pl.pallas_call(
        body,
        grid=(n // bn,),
        in_specs=[
            pl.BlockSpec((1, bn, h), lambda i: (0, i, 0)),
            pl.BlockSpec((1, bn, h), lambda i: (1, i, 0)),
            pl.BlockSpec((bn, h), lambda i: (i, 0)),
            pl.BlockSpec((bn, 16), lambda i: (i, 0)),
            pl.BlockSpec((1, h), lambda i: (0, 0)),
            pl.BlockSpec((h, cp), lambda i: (0, 0)),
        ],
        out_specs=pl.BlockSpec((bn, cp), lambda i: (i, 0)),
        out_shape=jax.ShapeDtypeStruct((n, cp), jnp.float32),
    )(p0, p1, g1, dinv16, b1, w2p)


def _tc_stage3(q0, q1, g2, dinv16, b2p, bn: int):
    n, cp = g2.shape

    def body(q0_ref, q1_ref, g2_ref, di_ref, b_ref, o_ref):
        di = di_ref[:, 0:1]
        s = q0_ref[0] + q1_ref[0] + g2_ref[...]
        o_ref[...] = s * di + b_ref[...]

    return pl.pallas_call(
        body,
        grid=(n // bn,),
        in_specs=[
            pl.BlockSpec((1, bn, cp), lambda i: (0, i, 0)),
            pl.BlockSpec((1, bn, cp), lambda i: (1, i, 0)),
            pl.BlockSpec((bn, cp), lambda i: (i, 0)),
            pl.BlockSpec((bn, 16), lambda i: (i, 0)),
            pl.BlockSpec((1, cp), lambda i: (0, 0)),
        ],
        out_specs=pl.BlockSpec((bn, cp), lambda i: (i, 0)),
        out_shape=jax.ShapeDtypeStruct((n, cp), jnp.float32),
    )(q0, q1, g2, dinv16, b2p)


def kernel(x, edge_index, W1, b1, W2, b2):
    n, dx = x.shape
    h = W1.shape[1]
    c = W2.shape[1]
    e = edge_index.shape[1]
    cp = _ceil_to(c, 16)
    bn = 1000
    assert n % bn == 0 and n % NS == 0

    src = edge_index[0]
    dst = edge_index[1]

    kc = _ceil_to(_ceil_to(e, NW * CH) // (NW * CH), 2)
    t2 = NW * kc * CH
    pad = t2 - e
    src_all = jnp.concatenate(
        [src, jnp.arange(pad, dtype=jnp.int32) % n]).reshape(NW, kc, CH)
    dst_all = jnp.concatenate(
        [dst, jnp.full((pad,), n, jnp.int32)]).reshape(NW, kc, CH)

    na = _ceil_to(n + 1, NS * 8)
    degp = _sc_count_rows(dst_all, na)

    g1, dinv16 = _tc_stage1(x, W1, degp, bn)

    p = _sc_scatter_rows(src_all, dst_all, g1, na, h)

    w2p = jnp.pad(W2, ((0, 0), (0, cp - c)))
    g2 = _tc_stage2(p, p, g1, dinv16, b1.reshape(1, h), w2p, bn)

    q = _sc_scatter_rows(src_all, dst_all, g2, na, cp)

    b2p = jnp.pad(b2, (0, cp - c)).reshape(1, cp)
    out = _tc_stage3(q, q, g2, dinv16, b2p, bn)

    return out[:, :c]


# speedup vs baseline: 7.2776x; 1.0091x over previous
import functools

import jax
import jax.numpy as jnp
from jax import lax
from jax.experimental import pallas as pl
from jax.experimental.pallas import tpu as pltpu
from jax.experimental.pallas import tpu_sc as plsc

NC = 2
NS = 16
NW = NC * NS
CH = 112


def _ceil_to(a: int, m: int) -> int:
    return ((a + m - 1) // m) * m


def _sc_scatter_rows(src_idx, dst_idx, table, n_out: int, d: int):
    kc = src_idx.shape[1]
    rpt = n_out // NS
    mesh = plsc.VectorSubcoreMesh(core_axis_name="c", subcore_axis_name="s")

    @functools.partial(
        pl.kernel,
        out_type=jax.ShapeDtypeStruct((NC, n_out, d), jnp.float32),
        mesh=mesh,
        scratch_types=[
            pltpu.VMEM((kc, CH), jnp.int32),
            pltpu.VMEM((kc, CH), jnp.int32),
            pltpu.VMEM((CH, d), jnp.float32),
            pltpu.VMEM((CH, d), jnp.float32),
            pltpu.VMEM_SHARED((n_out, d), jnp.float32),
            pltpu.SemaphoreType.DMA,
            pltpu.SemaphoreType.DMA,
        ],
        compiler_params=pltpu.CompilerParams(use_tc_tiling_on_sc=False),
    )
    def k(src_hbm, dst_hbm, tab_hbm, zero_hbm, out_hbm, sv, dv, ra, rb, acc,
          sa, sb):
        c = lax.axis_index("c")
        s = lax.axis_index("s")
        wid = c * NS + s
        pltpu.sync_copy(zero_hbm.at[pl.ds(s * rpt, rpt)],
                        acc.at[pl.ds(s * rpt, rpt)])
        pltpu.sync_copy(src_hbm.at[wid], sv)
        pltpu.sync_copy(dst_hbm.at[wid], dv)
        plsc.subcore_barrier()

        @pl.loop(0, kc, step=2)
        def _(j):
            pltpu.async_copy(tab_hbm.at[sv.at[j]], ra, sa)
            pltpu.async_copy(tab_hbm.at[sv.at[j + 1]], rb, sb)
            pltpu.make_async_copy(tab_hbm.at[sv.at[j]], ra, sa).wait()
            pltpu.make_async_copy(tab_hbm.at[sv.at[j + 1]], rb, sb).wait()
            pltpu.sync_copy(ra, acc.at[dv.at[j]], add=True)
            pltpu.sync_copy(rb, acc.at[dv.at[j + 1]], add=True)

        plsc.subcore_barrier()
        pltpu.sync_copy(acc.at[pl.ds(s * rpt, rpt)],
                        out_hbm.at[c, pl.ds(s * rpt, rpt)])

    zero = jnp.zeros((n_out, d), jnp.float32)
    return k(src_idx, dst_idx, table, zero)


def _sc_count_rows(dst_idx, n_out: int):
    d = 16
    kc = dst_idx.shape[1]
    rpt = n_out // NS
    mesh = plsc.VectorSubcoreMesh(core_axis_name="c", subcore_axis_name="s")

    @functools.partial(
        pl.kernel,
        out_type=jax.ShapeDtypeStruct((NC, n_out, d), jnp.float32),
        mesh=mesh,
        scratch_types=[
            pltpu.VMEM((kc, CH), jnp.int32),
            pltpu.VMEM((CH, d), jnp.float32),
            pltpu.VMEM_SHARED((n_out, d), jnp.float32),
            pltpu.SemaphoreType.DMA,
        ],
        compiler_params=pltpu.CompilerParams(use_tc_tiling_on_sc=False),
    )
    def k(dst_hbm, ones_hbm, zero_hbm, out_hbm, dv, ra, acc, sem):
        c = lax.axis_index("c")
        s = lax.axis_index("s")
        wid = c * NS + s
        pltpu.sync_copy(zero_hbm.at[pl.ds(s * rpt, rpt)],
                        acc.at[pl.ds(s * rpt, rpt)])
        pltpu.sync_copy(dst_hbm.at[wid], dv)
        pltpu.sync_copy(ones_hbm, ra)
        plsc.subcore_barrier()

        @pl.loop(0, kc)
        def _(j):
            pltpu.async_copy(ra, acc.at[dv.at[j]], sem, add=True)

        @pl.loop(0, kc)
        def _(j):
            pltpu.make_async_copy(ra, acc.at[dv.at[j]], sem).wait()

        plsc.subcore_barrier()
        pltpu.sync_copy(acc.at[pl.ds(s * rpt, rpt)],
                        out_hbm.at[c, pl.ds(s * rpt, rpt)])

    zero = jnp.zeros((n_out, d), jnp.float32)
    ones = jnp.ones((CH, d), jnp.float32)
    return k(dst_idx, ones, zero)


def _tc_stage1(x, w1, degp, bn: int):
    n, dx = x.shape
    h = w1.shape[1]

    def body(x_ref, w_ref, d0_ref, d1_ref, g_ref, di_ref):
        deg = d0_ref[0, :, 0:1] + d1_ref[0, :, 0:1] + 1.0
        dinv = lax.rsqrt(deg)
        hh = jnp.dot(x_ref[...], w_ref[...], preferred_element_type=jnp.float32)
        g_ref[...] = hh * dinv
        di_ref[...] = jnp.broadcast_to(dinv, di_ref.shape)

    return pl.pallas_call(
        body,
        grid=(n // bn,),
        in_specs=[
            pl.BlockSpec((bn, dx), lambda i: (i, 0)),
            pl.BlockSpec((dx, h), lambda i: (0, 0)),
            pl.BlockSpec((1, bn, 16), lambda i: (0, i, 0)),
            pl.BlockSpec((1, bn, 16), lambda i: (1, i, 0)),
        ],
        out_specs=[
            pl.BlockSpec((bn, h), lambda i: (i, 0)),
            pl.BlockSpec((bn, 16), lambda i: (i, 0)),
        ],
        out_shape=[
            jax.ShapeDtypeStruct((n, h), jnp.float32),
            jax.ShapeDtypeStruct((n, 16), jnp.float32),
        ],
    )(x, w1, degp, degp)


def _tc_stage2(p0, p1, g1, dinv16, b1, w2p, bn: int):
    n, h = g1.shape
    cp = w2p.shape[1]

    def body(p0_ref, p1_ref, g1_ref, di_ref, b_ref, w_ref, g_ref):
        di = di_ref[:, 0:1]
        s = p0_ref[0] + p1_ref[0] + g1_ref[...]
        a = jnp.maximum(s * di + b_ref[...], 0.0)
        hh = jnp.dot(a, w_ref[...], preferred_element_type=jnp.float32)
        g_ref[...] = hh * di

    return pl.pallas_call(
        body,
        grid=(n // bn,),
        in_specs=[
            pl.BlockSpec((1, bn, h), lambda i: (0, i, 0)),
            pl.BlockSpec((1, bn, h), lambda i: (1, i, 0)),
            pl.BlockSpec((bn, h), lambda i: (i, 0)),
            pl.BlockSpec((bn, 16), lambda i: (i, 0)),
            pl.BlockSpec((1, h), lambda i: (0, 0)),
            pl.BlockSpec((h, cp), lambda i: (0, 0)),
        ],
        out_specs=pl.BlockSpec((bn, cp), lambda i: (i, 0)),
        out_shape=jax.ShapeDtypeStruct((n, cp), jnp.float32),
    )(p0, p1, g1, dinv16, b1, w2p)


def _tc_stage3(q0, q1, g2, dinv16, b2p, bn: int):
    n, cp = g2.shape

    def body(q0_ref, q1_ref, g2_ref, di_ref, b_ref, o_ref):
        di = di_ref[:, 0:1]
        s = q0_ref[0] + q1_ref[0] + g2_ref[...]
        o_ref[...] = s * di + b_ref[...]

    return pl.pallas_call(
        body,
        grid=(n // bn,),
        in_specs=[
            pl.BlockSpec((1, bn, cp), lambda i: (0, i, 0)),
            pl.BlockSpec((1, bn, cp), lambda i: (1, i, 0)),
            pl.BlockSpec((bn, cp), lambda i: (i, 0)),
            pl.BlockSpec((bn, 16), lambda i: (i, 0)),
            pl.BlockSpec((1, cp), lambda i: (0, 0)),
        ],
        out_specs=pl.BlockSpec((bn, cp), lambda i: (i, 0)),
        out_shape=jax.ShapeDtypeStruct((n, cp), jnp.float32),
    )(q0, q1, g2, dinv16, b2p)


def kernel(x, edge_index, W1, b1, W2, b2):
    n, dx = x.shape
    h = W1.shape[1]
    c = W2.shape[1]
    e = edge_index.shape[1]
    cp = _ceil_to(c, 16)
    bn = 1000
    assert n % bn == 0 and n % NS == 0

    src = edge_index[0]
    dst = edge_index[1]

    kc = _ceil_to(_ceil_to(e, NW * CH) // (NW * CH), 2)
    t2 = NW * kc * CH
    pad = t2 - e
    src_all = jnp.concatenate(
        [src, jnp.arange(pad, dtype=jnp.int32) % n]).reshape(NW, kc, CH)
    dst_all = jnp.concatenate(
        [dst, jnp.full((pad,), n, jnp.int32)]).reshape(NW, kc, CH)

    na = _ceil_to(n + 1, NS * 8)
    degp = _sc_count_rows(dst_all, na)

    g1, dinv16 = _tc_stage1(x, W1, degp, bn)

    p = _sc_scatter_rows(src_all, dst_all, g1, na, h)

    w2p = jnp.pad(W2, ((0, 0), (0, cp - c)))
    g2 = _tc_stage2(p, p, g1, dinv16, b1.reshape(1, h), w2p, bn)

    q = _sc_scatter_rows(src_all, dst_all, g2, na, cp)

    b2p = jnp.pad(b2, (0, cp - c)).reshape(1, cp)
    out = _tc_stage3(q, q, g2, dinv16, b2p, bn)

    return out[:, :c]


# speedup vs baseline: 7.3898x; 1.0154x over previous
import functools

import jax
import jax.numpy as jnp
from jax import lax
from jax.experimental import pallas as pl
from jax.experimental.pallas import tpu as pltpu
from jax.experimental.pallas import tpu_sc as plsc

NC = 2
NS = 16
NW = NC * NS
CH = 112


def _ceil_to(a: int, m: int) -> int:
    return ((a + m - 1) // m) * m


def _sc_scatter_rows(src_idx, dst_idx, table, n_out: int, d: int):
    kc = src_idx.shape[1]
    rpt = n_out // NS
    mesh = plsc.VectorSubcoreMesh(core_axis_name="c", subcore_axis_name="s")

    @functools.partial(
        pl.kernel,
        out_type=jax.ShapeDtypeStruct((NC, n_out, d), jnp.float32),
        mesh=mesh,
        scratch_types=[
            pltpu.VMEM((kc, CH), jnp.int32),
            pltpu.VMEM((kc, CH), jnp.int32),
            pltpu.VMEM((CH, d), jnp.float32),
            pltpu.VMEM((CH, d), jnp.float32),
            pltpu.VMEM_SHARED((n_out, d), jnp.float32),
            pltpu.SemaphoreType.DMA,
            pltpu.SemaphoreType.DMA,
            pltpu.SemaphoreType.DMA,
            pltpu.SemaphoreType.DMA,
        ],
        compiler_params=pltpu.CompilerParams(use_tc_tiling_on_sc=False),
    )
    def k(src_hbm, dst_hbm, tab_hbm, zero_hbm, out_hbm, sv, dv, ra, rb, acc,
          sa, sb, wa, wb):
        c = lax.axis_index("c")
        s = lax.axis_index("s")
        wid = c * NS + s
        pltpu.sync_copy(zero_hbm.at[pl.ds(s * rpt, rpt)],
                        acc.at[pl.ds(s * rpt, rpt)])
        pltpu.sync_copy(src_hbm.at[wid], sv)
        pltpu.sync_copy(dst_hbm.at[wid], dv)
        plsc.subcore_barrier()

        @pl.loop(0, kc, step=2)
        def _(j):
            pltpu.async_copy(tab_hbm.at[sv.at[j]], ra, sa)
            pltpu.async_copy(tab_hbm.at[sv.at[j + 1]], rb, sb)
            pltpu.make_async_copy(tab_hbm.at[sv.at[j]], ra, sa).wait()
            pltpu.make_async_copy(tab_hbm.at[sv.at[j + 1]], rb, sb).wait()
            pltpu.async_copy(ra, acc.at[dv.at[j]], wa, add=True)
            pltpu.async_copy(rb, acc.at[dv.at[j + 1]], wb, add=True)
            pltpu.make_async_copy(ra, acc.at[dv.at[j]], wa).wait()
            pltpu.make_async_copy(rb, acc.at[dv.at[j + 1]], wb).wait()

        plsc.subcore_barrier()
        pltpu.sync_copy(acc.at[pl.ds(s * rpt, rpt)],
                        out_hbm.at[c, pl.ds(s * rpt, rpt)])

    zero = jnp.zeros((n_out, d), jnp.float32)
    return k(src_idx, dst_idx, table, zero)


def _sc_count_rows(dst_idx, n_out: int):
    d = 16
    kc = dst_idx.shape[1]
    rpt = n_out // NS
    mesh = plsc.VectorSubcoreMesh(core_axis_name="c", subcore_axis_name="s")

    @functools.partial(
        pl.kernel,
        out_type=jax.ShapeDtypeStruct((NC, n_out, d), jnp.float32),
        mesh=mesh,
        scratch_types=[
            pltpu.VMEM((kc, CH), jnp.int32),
            pltpu.VMEM((CH, d), jnp.float32),
            pltpu.VMEM_SHARED((n_out, d), jnp.float32),
            pltpu.SemaphoreType.DMA,
        ],
        compiler_params=pltpu.CompilerParams(use_tc_tiling_on_sc=False),
    )
    def k(dst_hbm, ones_hbm, zero_hbm, out_hbm, dv, ra, acc, sem):
        c = lax.axis_index("c")
        s = lax.axis_index("s")
        wid = c * NS + s
        pltpu.sync_copy(zero_hbm.at[pl.ds(s * rpt, rpt)],
                        acc.at[pl.ds(s * rpt, rpt)])
        pltpu.sync_copy(dst_hbm.at[wid], dv)
        pltpu.sync_copy(ones_hbm, ra)
        plsc.subcore_barrier()

        @pl.loop(0, kc)
        def _(j):
            pltpu.async_copy(ra, acc.at[dv.at[j]], sem, add=True)

        @pl.loop(0, kc)
        def _(j):
            pltpu.make_async_copy(ra, acc.at[dv.at[j]], sem).wait()

        plsc.subcore_barrier()
        pltpu.sync_copy(acc.at[pl.ds(s * rpt, rpt)],
                        out_hbm.at[c, pl.ds(s * rpt, rpt)])

    zero = jnp.zeros((n_out, d), jnp.float32)
    ones = jnp.ones((CH, d), jnp.float32)
    return k(dst_idx, ones, zero)


def _tc_stage1(x, w1, degp, bn: int):
    n, dx = x.shape
    h = w1.shape[1]

    def body(x_ref, w_ref, d0_ref, d1_ref, g_ref, di_ref):
        deg = d0_ref[0, :, 0:1] + d1_ref[0, :, 0:1] + 1.0
        dinv = lax.rsqrt(deg)
        hh = jnp.dot(x_ref[...], w_ref[...], preferred_element_type=jnp.float32)
        g_ref[...] = hh * dinv
        di_ref[...] = jnp.broadcast_to(dinv, di_ref.shape)

    return pl.pallas_call(
        body,
        grid=(n // bn,),
        in_specs=[
            pl.BlockSpec((bn, dx), lambda i: (i, 0)),
            pl.BlockSpec((dx, h), lambda i: (0, 0)),
            pl.BlockSpec((1, bn, 16), lambda i: (0, i, 0)),
            pl.BlockSpec((1, bn, 16), lambda i: (1, i, 0)),
        ],
        out_specs=[
            pl.BlockSpec((bn, h), lambda i: (i, 0)),
            pl.BlockSpec((bn, 16), lambda i: (i, 0)),
        ],
        out_shape=[
            jax.ShapeDtypeStruct((n, h), jnp.float32),
            jax.ShapeDtypeStruct((n, 16), jnp.float32),
        ],
    )(x, w1, degp, degp)


def _tc_stage2(p0, p1, g1, dinv16, b1, w2p, bn: int):
    n, h = g1.shape
    cp = w2p.shape[1]

    def body(p0_ref, p1_ref, g1_ref, di_ref, b_ref, w_ref, g_ref):
        di = di_ref[:, 0:1]
        s = p0_ref[0] + p1_ref[0] + g1_ref[...]
        a = jnp.maximum(s * di + b_ref[...], 0.0)
        hh = jnp.dot(a, w_ref[...], preferred_element_type=jnp.float32)
        g_ref[...] = hh * di

    return pl.pallas_call(
        body,
        grid=(n // bn,),
        in_specs=[
            pl.BlockSpec((1, bn, h), lambda i: (0, i, 0)),
            pl.BlockSpec((1, bn, h), lambda i: (1, i, 0)),
            pl.BlockSpec((bn, h), lambda i: (i, 0)),
            pl.BlockSpec((bn, 16), lambda i: (i, 0)),
            pl.BlockSpec((1, h), lambda i: (0, 0)),
            pl.BlockSpec((h, cp), lambda i: (0, 0)),
        ],
        out_specs=pl.BlockSpec((bn, cp), lambda i: (i, 0)),
        out_shape=jax.ShapeDtypeStruct((n, cp), jnp.float32),
    )(p0, p1, g1, dinv16, b1, w2p)


def _tc_stage3(q0, q1, g2, dinv16, b2p, bn: int):
    n, cp = g2.shape

    def body(q0_ref, q1_ref, g2_ref, di_ref, b_ref, o_ref):
        di = di_ref[:, 0:1]
        s = q0_ref[0] + q1_ref[0] + g2_ref[...]
        o_ref[...] = s * di + b_ref[...]

    return pl.pallas_call(
        body,
        grid=(n // bn,),
        in_specs=[
            pl.BlockSpec((1, bn, cp), lambda i: (0, i, 0)),
            pl.BlockSpec((1, bn, cp), lambda i: (1, i, 0)),
            pl.BlockSpec((bn, cp), lambda i: (i, 0)),
            pl.BlockSpec((bn, 16), lambda i: (i, 0)),
            pl.BlockSpec((1, cp), lambda i: (0, 0)),
        ],
        out_specs=pl.BlockSpec((bn, cp), lambda i: (i, 0)),
        out_shape=jax.ShapeDtypeStruct((n, cp), jnp.float32),
    )(q0, q1, g2, dinv16, b2p)


def kernel(x, edge_index, W1, b1, W2, b2):
    n, dx = x.shape
    h = W1.shape[1]
    c = W2.shape[1]
    e = edge_index.shape[1]
    cp = _ceil_to(c, 16)
    bn = 1000
    assert n % bn == 0 and n % NS == 0

    src = edge_index[0]
    dst = edge_index[1]

    kc = _ceil_to(_ceil_to(e, NW * CH) // (NW * CH), 2)
    t2 = NW * kc * CH
    pad = t2 - e
    src_all = jnp.concatenate(
        [src, jnp.arange(pad, dtype=jnp.int32) % n]).reshape(NW, kc, CH)
    dst_all = jnp.concatenate(
        [dst, jnp.full((pad,), n, jnp.int32)]).reshape(NW, kc, CH)

    na = _ceil_to(n + 1, NS * 8)
    degp = _sc_count_rows(dst_all, na)

    g1, dinv16 = _tc_stage1(x, W1, degp, bn)

    p = _sc_scatter_rows(src_all, dst_all, g1, na, h)

    w2p = jnp.pad(W2, ((0, 0), (0, cp - c)))
    g2 = _tc_stage2(p, p, g1, dinv16, b1.reshape(1, h), w2p, bn)

    q = _sc_scatter_rows(src_all, dst_all, g2, na, cp)

    b2p = jnp.pad(b2, (0, cp - c)).reshape(1, cp)
    out = _tc_stage3(q, q, g2, dinv16, b2p, bn)

    return out[:, :c]


# speedup vs baseline: 7.7640x; 1.0506x over previous
import functools

import jax
import jax.numpy as jnp
from jax import lax
from jax.experimental import pallas as pl
from jax.experimental.pallas import tpu as pltpu
from jax.experimental.pallas import tpu_sc as plsc

NC = 2
NS = 16
NW = NC * NS
CH = 56


def _ceil_to(a: int, m: int) -> int:
    return ((a + m - 1) // m) * m


def _sc_scatter_rows(src_idx, dst_idx, table, n_out: int, d: int):
    kc = src_idx.shape[1]
    rpt = n_out // NS
    mesh = plsc.VectorSubcoreMesh(core_axis_name="c", subcore_axis_name="s")

    @functools.partial(
        pl.kernel,
        out_type=jax.ShapeDtypeStruct((NC, n_out, d), jnp.float32),
        mesh=mesh,
        scratch_types=[
            pltpu.VMEM((kc, CH), jnp.int32),
            pltpu.VMEM((kc, CH), jnp.int32),
            pltpu.VMEM((CH, d), jnp.float32),
            pltpu.VMEM((CH, d), jnp.float32),
            pltpu.VMEM((CH, d), jnp.float32),
            pltpu.VMEM((CH, d), jnp.float32),
            pltpu.VMEM_SHARED((n_out, d), jnp.float32),
        ] + [pltpu.SemaphoreType.DMA] * 8,
        compiler_params=pltpu.CompilerParams(use_tc_tiling_on_sc=False),
    )
    def k(src_hbm, dst_hbm, tab_hbm, zero_hbm, out_hbm, sv, dv,
          ra, rb, rc, rd, acc, sa, sb, sc_, sd, wa, wb, wc, wd):
        c = lax.axis_index("c")
        s = lax.axis_index("s")
        wid = c * NS + s
        pltpu.sync_copy(zero_hbm.at[pl.ds(s * rpt, rpt)],
                        acc.at[pl.ds(s * rpt, rpt)])
        pltpu.sync_copy(src_hbm.at[wid], sv)
        pltpu.sync_copy(dst_hbm.at[wid], dv)
        plsc.subcore_barrier()

        pltpu.async_copy(tab_hbm.at[sv.at[0]], ra, sa)
        pltpu.async_copy(tab_hbm.at[sv.at[1]], rb, sb)

        @pl.loop(0, kc, step=4)
        def _(j):
            pltpu.make_async_copy(tab_hbm.at[sv.at[j]], ra, sa).wait()
            pltpu.make_async_copy(tab_hbm.at[sv.at[j + 1]], rb, sb).wait()
            pltpu.async_copy(ra, acc.at[dv.at[j]], wa, add=True)
            pltpu.async_copy(rb, acc.at[dv.at[j + 1]], wb, add=True)
            pltpu.async_copy(tab_hbm.at[sv.at[j + 2]], rc, sc_)
            pltpu.async_copy(tab_hbm.at[sv.at[j + 3]], rd, sd)
            pltpu.make_async_copy(ra, acc.at[dv.at[j]], wa).wait()
            pltpu.make_async_copy(rb, acc.at[dv.at[j + 1]], wb).wait()

            pltpu.make_async_copy(tab_hbm.at[sv.at[j + 2]], rc, sc_).wait()
            pltpu.make_async_copy(tab_hbm.at[sv.at[j + 3]], rd, sd).wait()
            pltpu.async_copy(rc, acc.at[dv.at[j + 2]], wc, add=True)
            pltpu.async_copy(rd, acc.at[dv.at[j + 3]], wd, add=True)

            @pl.when(j + 4 < kc)
            def _():
                pltpu.async_copy(tab_hbm.at[sv.at[j + 4]], ra, sa)
                pltpu.async_copy(tab_hbm.at[sv.at[j + 5]], rb, sb)

            pltpu.make_async_copy(rc, acc.at[dv.at[j + 2]], wc).wait()
            pltpu.make_async_copy(rd, acc.at[dv.at[j + 3]], wd).wait()

        plsc.subcore_barrier()
        pltpu.sync_copy(acc.at[pl.ds(s * rpt, rpt)],
                        out_hbm.at[c, pl.ds(s * rpt, rpt)])

    zero = jnp.zeros((n_out, d), jnp.float32)
    return k(src_idx, dst_idx, table, zero)


def _sc_count_rows(dst_idx, n_out: int):
    d = 16
    kc = dst_idx.shape[1]
    rpt = n_out // NS
    mesh = plsc.VectorSubcoreMesh(core_axis_name="c", subcore_axis_name="s")

    @functools.partial(
        pl.kernel,
        out_type=jax.ShapeDtypeStruct((NC, n_out, d), jnp.float32),
        mesh=mesh,
        scratch_types=[
            pltpu.VMEM((kc, CH), jnp.int32),
            pltpu.VMEM((CH, d), jnp.float32),
            pltpu.VMEM_SHARED((n_out, d), jnp.float32),
            pltpu.SemaphoreType.DMA,
        ],
        compiler_params=pltpu.CompilerParams(use_tc_tiling_on_sc=False),
    )
    def k(dst_hbm, ones_hbm, zero_hbm, out_hbm, dv, ra, acc, sem):
        c = lax.axis_index("c")
        s = lax.axis_index("s")
        wid = c * NS + s
        pltpu.sync_copy(zero_hbm.at[pl.ds(s * rpt, rpt)],
                        acc.at[pl.ds(s * rpt, rpt)])
        pltpu.sync_copy(dst_hbm.at[wid], dv)
        pltpu.sync_copy(ones_hbm, ra)
        plsc.subcore_barrier()

        @pl.loop(0, kc)
        def _(j):
            pltpu.async_copy(ra, acc.at[dv.at[j]], sem, add=True)

        @pl.loop(0, kc)
        def _(j):
            pltpu.make_async_copy(ra, acc.at[dv.at[j]], sem).wait()

        plsc.subcore_barrier()
        pltpu.sync_copy(acc.at[pl.ds(s * rpt, rpt)],
                        out_hbm.at[c, pl.ds(s * rpt, rpt)])

    zero = jnp.zeros((n_out, d), jnp.float32)
    ones = jnp.ones((CH, d), jnp.float32)
    return k(dst_idx, ones, zero)


def _tc_stage1(x, w1, degp, bn: int):
    n, dx = x.shape
    h = w1.shape[1]

    def body(x_ref, w_ref, d0_ref, d1_ref, g_ref, di_ref):
        deg = d0_ref[0, :, 0:1] + d1_ref[0, :, 0:1] + 1.0
        dinv = lax.rsqrt(deg)
        hh = jnp.dot(x_ref[...], w_ref[...], preferred_element_type=jnp.float32)
        g_ref[...] = hh * dinv
        di_ref[...] = jnp.broadcast_to(dinv, di_ref.shape)

    return pl.pallas_call(
        body,
        grid=(n // bn,),
        in_specs=[
            pl.BlockSpec((bn, dx), lambda i: (i, 0)),
            pl.BlockSpec((dx, h), lambda i: (0, 0)),
            pl.BlockSpec((1, bn, 16), lambda i: (0, i, 0)),
            pl.BlockSpec((1, bn, 16), lambda i: (1, i, 0)),
        ],
        out_specs=[
            pl.BlockSpec((bn, h), lambda i: (i, 0)),
            pl.BlockSpec((bn, 16), lambda i: (i, 0)),
        ],
        out_shape=[
            jax.ShapeDtypeStruct((n, h), jnp.float32),
            jax.ShapeDtypeStruct((n, 16), jnp.float32),
        ],
    )(x, w1, degp, degp)


def _tc_stage2(p0, p1, g1, dinv16, b1, w2p, bn: int):
    n, h = g1.shape
    cp = w2p.shape[1]

    def body(p0_ref, p1_ref, g1_ref, di_ref, b_ref, w_ref, g_ref):
        di = di_ref[:, 0:1]
        s = p0_ref[0] + p1_ref[0] + g1_ref[...]
        a = jnp.maximum(s * di + b_ref[...], 0.0)
        hh = jnp.dot(a, w_ref[...], preferred_element_type=jnp.float32)
        g_ref[...] = hh * di

    return pl.pallas_call(
        body,
        grid=(n // bn,),
        in_specs=[
            pl.BlockSpec((1, bn, h), lambda i: (0, i, 0)),
            pl.BlockSpec((1, bn, h), lambda i: (1, i, 0)),
            pl.BlockSpec((bn, h), lambda i: (i, 0)),
            pl.BlockSpec((bn, 16), lambda i: (i, 0)),
            pl.BlockSpec((1, h), lambda i: (0, 0)),
            pl.BlockSpec((h, cp), lambda i: (0, 0)),
        ],
        out_specs=pl.BlockSpec((bn, cp), lambda i: (i, 0)),
        out_shape=jax.ShapeDtypeStruct((n, cp), jnp.float32),
    )(p0, p1, g1, dinv16, b1, w2p)


def _tc_stage3(q0, q1, g2, dinv16, b2p, bn: int):
    n, cp = g2.shape

    def body(q0_ref, q1_ref, g2_ref, di_ref, b_ref, o_ref):
        di = di_ref[:, 0:1]
        s = q0_ref[0] + q1_ref[0] + g2_ref[...]
        o_ref[...] = s * di + b_ref[...]

    return pl.pallas_call(
        body,
        grid=(n // bn,),
        in_specs=[
            pl.BlockSpec((1, bn, cp), lambda i: (0, i, 0)),
            pl.BlockSpec((1, bn, cp), lambda i: (1, i, 0)),
            pl.BlockSpec((bn, cp), lambda i: (i, 0)),
            pl.BlockSpec((bn, 16), lambda i: (i, 0)),
            pl.BlockSpec((1, cp), lambda i: (0, 0)),
        ],
        out_specs=pl.BlockSpec((bn, cp), lambda i: (i, 0)),
        out_shape=jax.ShapeDtypeStruct((n, cp), jnp.float32),
    )(q0, q1, g2, dinv16, b2p)


def kernel(x, edge_index, W1, b1, W2, b2):
    n, dx = x.shape
    h = W1.shape[1]
    c = W2.shape[1]
    e = edge_index.shape[1]
    cp = _ceil_to(c, 16)
    bn = 1000
    assert n % bn == 0 and n % NS == 0

    src = edge_index[0]
    dst = edge_index[1]

    kc = _ceil_to(_ceil_to(e, NW * CH) // (NW * CH), 4)
    t2 = NW * kc * CH
    pad = t2 - e
    src_all = jnp.concatenate(
        [src, jnp.arange(pad, dtype=jnp.int32) % n]).reshape(NW, kc, CH)
    dst_all = jnp.concatenate(
        [dst, jnp.full((pad,), n, jnp.int32)]).reshape(NW, kc, CH)

    na = _ceil_to(n + 1, NS * 8)
    degp = _sc_count_rows(dst_all, na)

    g1, dinv16 = _tc_stage1(x, W1, degp, bn)

    p = _sc_scatter_rows(src_all, dst_all, g1, na, h)

    w2p = jnp.pad(W2, ((0, 0), (0, cp - c)))
    g2 = _tc_stage2(p, p, g1, dinv16, b1.reshape(1, h), w2p, bn)

    q = _sc_scatter_rows(src_all, dst_all, g2, na, cp)

    b2p = jnp.pad(b2, (0, cp - c)).reshape(1, cp)
    out = _tc_stage3(q, q, g2, dinv16, b2p, bn)

    return out[:, :c]
